# baseline (speedup 1.0000x reference)
import functools

import jax
import jax.numpy as jnp
from jax import lax
from jax.experimental import pallas as pl
from jax.experimental.pallas import tpu as pltpu
from jax.experimental.pallas import tpu_sc as plsc

_TOPK = 8
_N_GROUPS = 8
_TOPK_GROUPS = 4
_ROUTE_SCALE = 2.5
_NEG = -1e30
_L = 16


def _matmul_kernel(x_ref, w_ref, s_ref):
    z = jax.lax.dot_general(
        w_ref[...], x_ref[...],
        dimension_numbers=(((1,), (1,)), ((), ())),
        preferred_element_type=jnp.float32)
    s_ref[...] = 1.0 / (1.0 + jnp.exp(-z))


def _scores_tc(x, weight):
    t, d = x.shape
    e = weight.shape[0]
    tb = 2048
    while tb > 8 and t % tb != 0:
        tb //= 2
    nt = t // tb
    return pl.pallas_call(
        _matmul_kernel,
        grid=(nt,),
        in_specs=[
            pl.BlockSpec((tb, d), lambda i: (i, 0)),
            pl.BlockSpec((e, d), lambda i: (0, 0)),
        ],
        out_specs=pl.BlockSpec((e, tb), lambda i: (0, i)),
        out_shape=jax.ShapeDtypeStruct((e, t), jnp.float32),
        compiler_params=pltpu.CompilerParams(
            dimension_semantics=("parallel",)),
    )(x, weight)


def _make_sc_route(t, e, nw, nc, tpw):
    mesh = plsc.VectorSubcoreMesh(core_axis_name="c", subcore_axis_name="s")

    @functools.partial(
        pl.kernel,
        mesh=mesh,
        out_type=[
            jax.ShapeDtypeStruct((_TOPK, t), jnp.float32),
            jax.ShapeDtypeStruct((_TOPK, t), jnp.int32),
        ],
        scratch_types=[
            pltpu.VMEM((e, tpw), jnp.float32),
            pltpu.VMEM((_TOPK, tpw), jnp.float32),
            pltpu.VMEM((_TOPK, tpw), jnp.int32),
        ],
    )
    def sc_route(s_hbm, wout_hbm, iout_hbm, sv, wv, iv):
        wid = lax.axis_index("s") * nc + lax.axis_index("c")
        base = wid * tpw
        pltpu.sync_copy(s_hbm.at[:, pl.ds(base, tpw)], sv)

        def chunk(c, carry):
            econst = [jnp.full((_L,), k, jnp.int32) for k in range(e)]
            neg = jnp.full((_L,), _NEG, jnp.float32)
            off = c * _L
            v = [sv[k, pl.ds(off, _L)] for k in range(e)]

            gs = []
            for g in range(_N_GROUPS):
                m1 = v[8 * g]
                m2 = neg
                for j in range(1, 8):
                    nv = v[8 * g + j]
                    m2 = jnp.maximum(m2, jnp.minimum(m1, nv))
                    m1 = jnp.maximum(m1, nv)
                gs.append(m1 + m2)

            rank = [jnp.full((_L,), 0, jnp.int32) for _ in range(_N_GROUPS)]
            for a in range(_N_GROUPS):
                for b in range(a + 1, _N_GROUPS):
                    one = jnp.full((_L,), 1, jnp.int32)
                    zero = jnp.full((_L,), 0, jnp.int32)
                    rank[b] = rank[b] + jnp.where(gs[a] >= gs[b], one, zero)
                    rank[a] = rank[a] + jnp.where(gs[b] > gs[a], one, zero)
            selg = [rank[g] < _TOPK_GROUPS for g in range(_N_GROUPS)]

            mv = [jnp.where(selg[k // 8], v[k], neg) for k in range(e)]

            ws = []
            wsum = jnp.full((_L,), 0.0, jnp.float32)
            for r in range(_TOPK):
                cv = list(mv)
                ci = list(econst)
                while len(cv) > 1:
                    nv2, ni2 = [], []
                    for p in range(0, len(cv), 2):
                        keep = cv[p] >= cv[p + 1]
                        nv2.append(jnp.where(keep, cv[p], cv[p + 1]))
                        ni2.append(jnp.where(keep, ci[p], ci[p + 1]))
                    cv, ci = nv2, ni2
                m, bi = cv[0], ci[0]
                ws.append(m)
                wsum = wsum + m
                iv[r, pl.ds(off, _L)] = bi
                mv = [jnp.where(bi == econst[k], neg, mv[k])
                      for k in range(e)]

            scale = jnp.full((_L,), _ROUTE_SCALE, jnp.float32) / wsum
            for r in range(_TOPK):
                wv[r, pl.ds(off, _L)] = ws[r] * scale
            return carry

        lax.fori_loop(0, tpw // _L, chunk, 0)

        pltpu.sync_copy(wv, wout_hbm.at[:, pl.ds(base, tpw)])
        pltpu.sync_copy(iv, iout_hbm.at[:, pl.ds(base, tpw)])

    return sc_route


@functools.partial(jax.jit, static_argnames=())
def kernel(x, weight):
    t, d = x.shape
    e = weight.shape[0]
    scores = _scores_tc(x, weight)
    info = plsc.get_sparse_core_info()
    nc, ns = info.num_cores, info.num_subcores
    nw = nc * ns
    tpw = t // nw
    w8, i8 = _make_sc_route(t, e, nw, nc, tpw)(scores)
    return w8.T.astype(x.dtype), i8.T


# speedup vs baseline: 1.9472x; 1.9472x over previous
import functools

import jax
import jax.numpy as jnp
from jax.experimental import pallas as pl
from jax.experimental.pallas import tpu as pltpu

_TOPK = 8
_N_GROUPS = 8
_TOPK_GROUPS = 4
_ROUTE_SCALE = 2.5
_NEG = -1e30
_CW = 512


def _route_chunk(s, c0, cw, wout_ref, iout_ref):
    g = s.reshape(_N_GROUPS, 8, cw)
    m1 = jnp.max(g, axis=1)
    eq = g == m1[:, None, :]
    cnt = jnp.sum(eq.astype(jnp.float32), axis=1)
    m2 = jnp.where(cnt >= 2.0, m1,
                   jnp.max(jnp.where(eq, _NEG, g), axis=1))
    gw = m1 + m2

    giota = jax.lax.broadcasted_iota(jnp.int32, (_N_GROUPS, cw), 0)
    selg = giota >= _N_GROUPS
    for _ in range(_TOPK_GROUPS):
        gm = jnp.max(gw, axis=0, keepdims=True)
        bi = jnp.min(jnp.where(gw == gm, giota, _N_GROUPS),
                     axis=0, keepdims=True)
        hit = giota == bi
        selg = selg | hit
        gw = jnp.where(hit, _NEG, gw)
    sel = jnp.broadcast_to(selg[:, None, :], (_N_GROUPS, 8, cw))
    masked = jnp.where(sel.reshape(64, cw), s, _NEG)

    eio = jax.lax.broadcasted_iota(jnp.int32, (64, cw), 0)
    wsum = jnp.zeros((1, cw), jnp.float32)
    for r in range(_TOPK):
        m = jnp.max(masked, axis=0, keepdims=True)
        bi = jnp.min(jnp.where(masked == m, eio, 64),
                     axis=0, keepdims=True)
        wout_ref[pl.ds(r, 1), pl.ds(c0, cw)] = m
        iout_ref[pl.ds(r, 1), pl.ds(c0, cw)] = bi
        wsum = wsum + m
        masked = jnp.where(eio == bi, _NEG, masked)

    wout_ref[:, pl.ds(c0, cw)] = (
        wout_ref[:, pl.ds(c0, cw)] * (_ROUTE_SCALE / wsum))


def _gate_kernel(x_ref, w_ref, wout_ref, iout_ref, sbuf_ref):
    tb = x_ref.shape[0]
    cw = min(_CW, tb)

    for c in range(tb // cw):
        _route_chunk(sbuf_ref[:, pl.ds(c * cw, cw)], c * cw, cw,
                     wout_ref, iout_ref)

    z = jax.lax.dot_general(
        w_ref[...], x_ref[...],
        dimension_numbers=(((1,), (1,)), ((), ())),
        preferred_element_type=jnp.float32)
    sbuf_ref[...] = 1.0 / (1.0 + jnp.exp(-z))


@functools.partial(jax.jit, static_argnames=())
def kernel(x, weight):
    t, d = x.shape
    e = weight.shape[0]
    tb = 2048
    while tb > 8 and t % tb != 0:
        tb //= 2
    nt = t // tb
    w8, i8 = pl.pallas_call(
        _gate_kernel,
        grid=(nt + 1,),
        in_specs=[
            pl.BlockSpec((tb, d), lambda i: (jnp.minimum(i, nt - 1), 0)),
            pl.BlockSpec((e, d), lambda i: (0, 0)),
        ],
        out_specs=[
            pl.BlockSpec((_TOPK, tb), lambda i: (0, jnp.maximum(i - 1, 0))),
            pl.BlockSpec((_TOPK, tb), lambda i: (0, jnp.maximum(i - 1, 0))),
        ],
        out_shape=[
            jax.ShapeDtypeStruct((_TOPK, t), jnp.float32),
            jax.ShapeDtypeStruct((_TOPK, t), jnp.int32),
        ],
        scratch_shapes=[pltpu.VMEM((e, tb), jnp.float32)],
        compiler_params=pltpu.CompilerParams(
            dimension_semantics=("arbitrary",)),
    )(x, weight)
    return w8.T.astype(x.dtype), i8.T


# speedup vs baseline: 1.9959x; 1.0250x over previous
import functools

import jax
import jax.numpy as jnp
from jax.experimental import pallas as pl
from jax.experimental.pallas import tpu as pltpu

_TOPK = 8
_N_GROUPS = 8
_TOPK_GROUPS = 4
_ROUTE_SCALE = 2.5
_NEG = -1e30
_CW = 512


def _route_chunk(s, c0, cw, wout_ref, iout_ref):
    g = s.reshape(_N_GROUPS, 8, cw)
    a1 = jnp.maximum(g[:, :4], g[:, 4:])
    a2 = jnp.minimum(g[:, :4], g[:, 4:])
    b1 = jnp.maximum(a1[:, :2], a1[:, 2:])
    b2 = jnp.maximum(jnp.minimum(a1[:, :2], a1[:, 2:]),
                     jnp.maximum(a2[:, :2], a2[:, 2:]))
    c1 = jnp.maximum(b1[:, 0], b1[:, 1])
    c2 = jnp.maximum(jnp.minimum(b1[:, 0], b1[:, 1]),
                     jnp.maximum(b2[:, 0], b2[:, 1]))
    gw = c1 + c2

    giota = jax.lax.broadcasted_iota(jnp.int32, (_N_GROUPS, cw), 0)
    selg = giota >= _N_GROUPS
    for _ in range(_TOPK_GROUPS):
        gm = jnp.max(gw, axis=0, keepdims=True)
        bi = jnp.min(jnp.where(gw == gm, giota, _N_GROUPS),
                     axis=0, keepdims=True)
        hit = giota == bi
        selg = selg | hit
        gw = jnp.where(hit, _NEG, gw)
    sel = jnp.broadcast_to(selg[:, None, :], (_N_GROUPS, 8, cw))
    masked = jnp.where(sel.reshape(64, cw), s, _NEG)

    eio = jax.lax.broadcasted_iota(jnp.int32, (64, cw), 0)
    wsum = jnp.zeros((1, cw), jnp.float32)
    for r in range(_TOPK):
        m = jnp.max(masked, axis=0, keepdims=True)
        bi = jnp.min(jnp.where(masked == m, eio, 64),
                     axis=0, keepdims=True)
        wout_ref[pl.ds(r, 1), pl.ds(c0, cw)] = m
        iout_ref[pl.ds(r, 1), pl.ds(c0, cw)] = bi
        wsum = wsum + m
        masked = jnp.where(eio == bi, _NEG, masked)

    wout_ref[:, pl.ds(c0, cw)] = (
        wout_ref[:, pl.ds(c0, cw)] * (_ROUTE_SCALE / wsum))


def _gate_kernel(x_ref, w_ref, wout_ref, iout_ref, sbuf_ref):
    tb = x_ref.shape[0]
    cw = min(_CW, tb)

    for c in range(tb // cw):
        _route_chunk(sbuf_ref[:, pl.ds(c * cw, cw)], c * cw, cw,
                     wout_ref, iout_ref)

    z = jax.lax.dot_general(
        w_ref[...], x_ref[...],
        dimension_numbers=(((1,), (1,)), ((), ())),
        preferred_element_type=jnp.float32)
    sbuf_ref[...] = 1.0 / (1.0 + jnp.exp(-z))


@functools.partial(jax.jit, static_argnames=())
def kernel(x, weight):
    t, d = x.shape
    e = weight.shape[0]
    tb = 2048
    while tb > 8 and t % tb != 0:
        tb //= 2
    nt = t // tb
    w8, i8 = pl.pallas_call(
        _gate_kernel,
        grid=(nt + 1,),
        in_specs=[
            pl.BlockSpec((tb, d), lambda i: (jnp.minimum(i, nt - 1), 0)),
            pl.BlockSpec((e, d), lambda i: (0, 0)),
        ],
        out_specs=[
            pl.BlockSpec((_TOPK, tb), lambda i: (0, jnp.maximum(i - 1, 0))),
            pl.BlockSpec((_TOPK, tb), lambda i: (0, jnp.maximum(i - 1, 0))),
        ],
        out_shape=[
            jax.ShapeDtypeStruct((_TOPK, t), jnp.float32),
            jax.ShapeDtypeStruct((_TOPK, t), jnp.int32),
        ],
        scratch_shapes=[pltpu.VMEM((e, tb), jnp.float32)],
        compiler_params=pltpu.CompilerParams(
            dimension_semantics=("arbitrary",)),
    )(x, weight)
    return w8.T.astype(x.dtype), i8.T
